# baseline (speedup 1.0000x reference)
import jax
import jax.numpy as jnp
from jax.experimental import pallas as pl
from jax.experimental.pallas import tpu as pltpu

_BB = 128
_NSTEP = 8


def _body(idx_ref, x_ref, y_ref, o_ref, buf0, buf1, ybuf, is0, is1, os0, os1, ysem):
    bufs = (buf0, buf1)
    isems = (is0, is1)
    osems = (os0, os1)
    idx = idx_ref[0]

    def in_copy(k):
        return pltpu.make_async_copy(
            x_ref.at[pl.ds(k * _BB, _BB)], bufs[k % 2], isems[k % 2]
        )

    def out_copy(k):
        return pltpu.make_async_copy(
            bufs[k % 2], o_ref.at[pl.ds(k * _BB, _BB)], osems[k % 2]
        )

    ydma = pltpu.make_async_copy(y_ref, ybuf, ysem)
    ydma.start()
    in_copy(0).start()
    ydma.wait()
    for k in range(_NSTEP):
        in_copy(k).wait()
        bufs[k % 2][:, pl.ds(idx, 1), :] = ybuf[pl.ds(k * _BB, _BB), :][:, None, :]
        out_copy(k).start()
        if k + 1 < _NSTEP:
            if k >= 1:
                out_copy(k - 1).wait()
            in_copy(k + 1).start()
    out_copy(_NSTEP - 2).wait()
    out_copy(_NSTEP - 1).wait()


def kernel(x, y, dim, index):
    del dim
    n, s, d = x.shape
    idx = jnp.reshape(jnp.asarray(index, jnp.int32), (1,))
    grid_spec = pltpu.PrefetchScalarGridSpec(
        num_scalar_prefetch=1,
        grid=(1,),
        in_specs=[
            pl.BlockSpec(memory_space=pltpu.MemorySpace.HBM),
            pl.BlockSpec(memory_space=pltpu.MemorySpace.HBM),
        ],
        out_specs=pl.BlockSpec(memory_space=pltpu.MemorySpace.HBM),
        scratch_shapes=[
            pltpu.VMEM((_BB, s, d), jnp.float32),
            pltpu.VMEM((_BB, s, d), jnp.float32),
            pltpu.VMEM((n, d), jnp.float32),
            pltpu.SemaphoreType.DMA,
            pltpu.SemaphoreType.DMA,
            pltpu.SemaphoreType.DMA,
            pltpu.SemaphoreType.DMA,
            pltpu.SemaphoreType.DMA,
        ],
    )
    return pl.pallas_call(
        _body,
        grid_spec=grid_spec,
        out_shape=jax.ShapeDtypeStruct((n, s, d), x.dtype),
    )(idx, x, y)


# speedup vs baseline: 1.1444x; 1.1444x over previous
import jax
import jax.numpy as jnp
from jax.experimental import pallas as pl
from jax.experimental.pallas import tpu as pltpu

_BB = 64
_NBUF = 4
_NSTEP = 16


def _body(idx_ref, x_ref, y_ref, o_ref, b0, b1, b2, b3, ybuf, *sems):
    bufs = (b0, b1, b2, b3)
    isems = sems[:_NBUF]
    osems = sems[_NBUF:2 * _NBUF]
    ysem = sems[2 * _NBUF]
    idx = idx_ref[0]

    def in_copy(k):
        return pltpu.make_async_copy(
            x_ref.at[pl.ds(k * _BB, _BB)], bufs[k % _NBUF], isems[k % _NBUF]
        )

    def out_copy(k):
        return pltpu.make_async_copy(
            bufs[k % _NBUF], o_ref.at[pl.ds(k * _BB, _BB)], osems[k % _NBUF]
        )

    ydma = pltpu.make_async_copy(y_ref, ybuf, ysem)
    ydma.start()
    in_copy(0).start()
    in_copy(1).start()
    in_copy(2).start()
    ydma.wait()
    for k in range(_NSTEP):
        in_copy(k).wait()
        bufs[k % _NBUF][:, pl.ds(idx, 1), :] = (
            ybuf[pl.ds(k * _BB, _BB), :][:, None, :]
        )
        out_copy(k).start()
        if k + 3 < _NSTEP:
            if k >= 1:
                out_copy(k - 1).wait()
            in_copy(k + 3).start()
    for k in range(_NSTEP - _NBUF, _NSTEP):
        out_copy(k).wait()


def kernel(x, y, dim, index):
    del dim
    n, s, d = x.shape
    idx = jnp.reshape(jnp.asarray(index, jnp.int32), (1,))
    grid_spec = pltpu.PrefetchScalarGridSpec(
        num_scalar_prefetch=1,
        grid=(1,),
        in_specs=[
            pl.BlockSpec(memory_space=pltpu.MemorySpace.HBM),
            pl.BlockSpec(memory_space=pltpu.MemorySpace.HBM),
        ],
        out_specs=pl.BlockSpec(memory_space=pltpu.MemorySpace.HBM),
        scratch_shapes=(
            [pltpu.VMEM((_BB, s, d), jnp.float32)] * _NBUF
            + [pltpu.VMEM((n, d), jnp.float32)]
            + [pltpu.SemaphoreType.DMA] * (2 * _NBUF + 1)
        ),
    )
    return pl.pallas_call(
        _body,
        grid_spec=grid_spec,
        out_shape=jax.ShapeDtypeStruct((n, s, d), x.dtype),
    )(idx, x, y)


# speedup vs baseline: 1.1465x; 1.0018x over previous
import jax
import jax.numpy as jnp
from jax.experimental import pallas as pl
from jax.experimental.pallas import tpu as pltpu

_BB = 128


def _body(idx_ref, x_ref, y_ref, o_ref):
    o_ref[...] = x_ref[...]
    idx = idx_ref[0]
    o_ref[:, pl.ds(idx, 1), :] = y_ref[...][:, None, :]


def kernel(x, y, dim, index):
    del dim
    n, s, d = x.shape
    idx = jnp.reshape(jnp.asarray(index, jnp.int32), (1,))
    grid_spec = pltpu.PrefetchScalarGridSpec(
        num_scalar_prefetch=1,
        grid=(n // _BB,),
        in_specs=[
            pl.BlockSpec((_BB, s, d), lambda i, idx_ref: (i, 0, 0)),
            pl.BlockSpec((_BB, d), lambda i, idx_ref: (i, 0)),
        ],
        out_specs=pl.BlockSpec((_BB, s, d), lambda i, idx_ref: (i, 0, 0)),
    )
    return pl.pallas_call(
        _body,
        grid_spec=grid_spec,
        out_shape=jax.ShapeDtypeStruct((n, s, d), x.dtype),
        compiler_params=pltpu.CompilerParams(
            dimension_semantics=("parallel",),
        ),
    )(idx, x, y)
